# Initial kernel scaffold; baseline (speedup 1.0000x reference)
import jax
import jax.numpy as jnp
from jax.experimental import pallas as pl


def kernel(x, Wq, Wk, Wv, Wo, sel_v_w, sel_o_w):
    raise NotImplementedError("write your pallas kernel here")


# speedup vs baseline: 1.4648x; 1.4648x over previous
import functools
import math

import jax
import jax.numpy as jnp
import numpy as np
from jax import lax
from jax.experimental import pallas as pl
from jax.experimental.pallas import tpu as pltpu

D_MODEL_C = 768
N_HEADS_C = 12
N_EXPERTS_C = 8
D_HEAD_C = 64
HE_C = N_HEADS_C * N_EXPERTS_C
SEQ_C = 2048
SBLK = 256

_F32 = jnp.float32


def _perm_matrix_np():
    P = np.zeros((HE_C, 7 * HE_C), dtype=np.float32)
    for d in range(1, 8):
        for h in range(N_HEADS_C):
            for e in range(N_EXPERTS_C):
                src = h * 8 + (e + d) % 8
                dst = (d - 1) * HE_C + h * 8 + e
                P[src, dst] = 1.0
    return P


_PERM_NP = _perm_matrix_np()


def _top2_gates(logits, p_mat):
    g = jax.nn.sigmoid(logits)
    rows = g.shape[0]
    p_all = lax.dot_general(g, p_mat, (((1,), (0,)), ((), ())),
                            precision=lax.Precision.HIGHEST,
                            preferred_element_type=_F32)
    e_idx = lax.broadcasted_iota(jnp.int32, (rows, HE_C), 1) & 7
    rank = jnp.zeros((rows, HE_C), dtype=jnp.int32)
    for d in range(1, 8):
        p = p_all[:, (d - 1) * HE_C:d * HE_C]
        partner_e = (e_idx + d) & 7
        beats = (p > g) | ((p == g) & (partner_e < e_idx))
        rank = rank + beats.astype(jnp.int32)
    return g * (rank < 2).astype(_F32)


def _proj_gates_body(x_ref, wq_ref, wk_ref, svw_ref, sow_ref, p_ref,
                     q_ref, k_ref, gv_ref, go_ref):
    xb = x_ref[...]
    qb = lax.dot_general(xb, wq_ref[...], (((1,), (1,)), ((), ())),
                         preferred_element_type=_F32)
    kb = lax.dot_general(xb, wk_ref[...], (((1,), (1,)), ((), ())),
                         preferred_element_type=_F32)
    sv = lax.dot_general(xb, svw_ref[...], (((1,), (1,)), ((), ())),
                         preferred_element_type=_F32)
    so = lax.dot_general(xb, sow_ref[...], (((1,), (1,)), ((), ())),
                         preferred_element_type=_F32)
    p_mat = p_ref[...]
    gvb = _top2_gates(sv, p_mat)
    go_ref[...] = _top2_gates(so, p_mat)
    for h in range(N_HEADS_C):
        q_ref[h] = qb[:, h * D_HEAD_C:(h + 1) * D_HEAD_C]
        k_ref[h] = kb[:, h * D_HEAD_C:(h + 1) * D_HEAD_C]
        gv_ref[h] = gvb[:, h * 8:(h + 1) * 8]


def _v_body(x_ref, wv_ref, gv_ref, v_ref):
    gvh = gv_ref[0]
    for si in range(SEQ_C // SBLK):
        xb = x_ref[si * SBLK:(si + 1) * SBLK, :]
        acc = jnp.zeros((SBLK, D_HEAD_C), dtype=_F32)
        for e in range(N_EXPERTS_C):
            pe = lax.dot_general(xb, wv_ref[e], (((1,), (0,)), ((), ())),
                                 preferred_element_type=_F32)
            gcol = gvh[si * SBLK:(si + 1) * SBLK, e:e + 1]
            acc = acc + gcol * pe
        v_ref[0, si * SBLK:(si + 1) * SBLK, :] = acc


def _attn_body(q_ref, k_ref, v_ref, o_ref, m_sc, l_sc, acc_sc):
    qi = pl.program_id(1)
    kj = pl.program_id(2)
    nk = pl.num_programs(2)
    scale = 1.0 / math.sqrt(D_HEAD_C)

    @pl.when(kj == 0)
    def _init():
        m_sc[...] = jnp.full((SBLK, 1), -1e30, dtype=_F32)
        l_sc[...] = jnp.zeros((SBLK, 1), dtype=_F32)
        acc_sc[...] = jnp.zeros((SBLK, D_HEAD_C), dtype=_F32)

    @pl.when(kj <= qi)
    def _compute():
        q = q_ref[0] * scale
        kb = k_ref[0]
        vb = v_ref[0]
        s = lax.dot_general(q, kb, (((1,), (1,)), ((), ())),
                            preferred_element_type=_F32)
        row = qi * SBLK + lax.broadcasted_iota(jnp.int32, (SBLK, SBLK), 0)
        col = kj * SBLK + lax.broadcasted_iota(jnp.int32, (SBLK, SBLK), 1)
        s = jnp.where(col <= row, s, -1e30)
        m = m_sc[...]
        mn = jnp.maximum(m, jnp.max(s, axis=1, keepdims=True))
        p = jnp.exp(s - mn)
        corr = jnp.exp(m - mn)
        m_sc[...] = mn
        l_sc[...] = l_sc[...] * corr + jnp.sum(p, axis=1, keepdims=True)
        acc_sc[...] = acc_sc[...] * corr + lax.dot_general(
            p, vb, (((1,), (0,)), ((), ())), preferred_element_type=_F32)

    @pl.when(kj == nk - 1)
    def _fin():
        o_ref[0] = acc_sc[...] / l_sc[...]


def _out_body(res_ref, go_ref, wo_ref, out_ref, scratch):
    go = go_ref[...]
    for h in range(N_HEADS_C):
        rh = res_ref[h]
        for e in range(N_EXPERTS_C):
            he = h * 8 + e
            scratch[:, he * D_HEAD_C:(he + 1) * D_HEAD_C] = (
                go[:, he:he + 1] * rh)
    out_ref[...] = lax.dot_general(scratch[...], wo_ref[...],
                                   (((1,), (0,)), ((), ())),
                                   preferred_element_type=_F32)


@jax.jit
def kernel(x, Wq, Wk, Wv, Wo, sel_v_w, sel_o_w):
    B, S, D = x.shape
    x2d = x.reshape(S, D)
    p_mat = jnp.asarray(_PERM_NP)

    nsb = S // SBLK
    H, E, dh = N_HEADS_C, N_EXPERTS_C, D_HEAD_C

    q, k, gv, go = pl.pallas_call(
        _proj_gates_body,
        grid=(nsb,),
        in_specs=[
            pl.BlockSpec((SBLK, D), lambda i: (i, 0)),
            pl.BlockSpec((D, D), lambda i: (0, 0)),
            pl.BlockSpec((D, D), lambda i: (0, 0)),
            pl.BlockSpec((HE_C, D), lambda i: (0, 0)),
            pl.BlockSpec((HE_C, D), lambda i: (0, 0)),
            pl.BlockSpec((HE_C, 7 * HE_C), lambda i: (0, 0)),
        ],
        out_specs=[
            pl.BlockSpec((H, SBLK, dh), lambda i: (0, i, 0)),
            pl.BlockSpec((H, SBLK, dh), lambda i: (0, i, 0)),
            pl.BlockSpec((H, SBLK, E), lambda i: (0, i, 0)),
            pl.BlockSpec((SBLK, HE_C), lambda i: (i, 0)),
        ],
        out_shape=[
            jax.ShapeDtypeStruct((H, S, dh), _F32),
            jax.ShapeDtypeStruct((H, S, dh), _F32),
            jax.ShapeDtypeStruct((H, S, E), _F32),
            jax.ShapeDtypeStruct((S, HE_C), _F32),
        ],
    )(x2d, Wq, Wk, sel_v_w, sel_o_w, p_mat)

    v = pl.pallas_call(
        _v_body,
        grid=(H,),
        in_specs=[
            pl.BlockSpec((S, D), lambda h: (0, 0)),
            pl.BlockSpec((E, D, dh), lambda h: (h, 0, 0)),
            pl.BlockSpec((1, S, E), lambda h: (h, 0, 0)),
        ],
        out_specs=pl.BlockSpec((1, S, dh), lambda h: (h, 0, 0)),
        out_shape=jax.ShapeDtypeStruct((H, S, dh), _F32),
    )(x2d, Wv, gv)

    res = pl.pallas_call(
        _attn_body,
        grid=(H, nsb, nsb),
        in_specs=[
            pl.BlockSpec((1, SBLK, dh), lambda h, qi, kj: (h, qi, 0)),
            pl.BlockSpec((1, SBLK, dh),
                         lambda h, qi, kj: (h, jnp.minimum(kj, qi), 0)),
            pl.BlockSpec((1, SBLK, dh),
                         lambda h, qi, kj: (h, jnp.minimum(kj, qi), 0)),
        ],
        out_specs=pl.BlockSpec((1, SBLK, dh), lambda h, qi, kj: (h, qi, 0)),
        out_shape=jax.ShapeDtypeStruct((H, S, dh), _F32),
        scratch_shapes=[
            pltpu.VMEM((SBLK, 1), _F32),
            pltpu.VMEM((SBLK, 1), _F32),
            pltpu.VMEM((SBLK, D_HEAD_C), _F32),
        ],
    )(q, k, v)

    wo2d = Wo.reshape(HE_C * dh, D)
    out2d = pl.pallas_call(
        _out_body,
        grid=(nsb,),
        in_specs=[
            pl.BlockSpec((H, SBLK, dh), lambda i: (0, i, 0)),
            pl.BlockSpec((SBLK, HE_C), lambda i: (i, 0)),
            pl.BlockSpec((HE_C * dh, D), lambda i: (0, 0)),
        ],
        out_specs=pl.BlockSpec((SBLK, D), lambda i: (i, 0)),
        out_shape=jax.ShapeDtypeStruct((S, D), _F32),
        scratch_shapes=[pltpu.VMEM((SBLK, HE_C * dh), _F32)],
    )(res, go, wo2d)

    return out2d.reshape(B, S, D)


# speedup vs baseline: 2.6804x; 1.8299x over previous
import functools
import math

import jax
import jax.numpy as jnp
import numpy as np
from jax import lax
from jax.experimental import pallas as pl
from jax.experimental.pallas import tpu as pltpu

D_MODEL_C = 768
N_HEADS_C = 12
N_EXPERTS_C = 8
D_HEAD_C = 64
HE_C = N_HEADS_C * N_EXPERTS_C
SEQ_C = 2048
SBLK = 256

_F32 = jnp.float32


def _perm_matrix_np():
    P = np.zeros((HE_C, 7 * HE_C), dtype=np.float32)
    for d in range(1, 8):
        for h in range(N_HEADS_C):
            for e in range(N_EXPERTS_C):
                src = h * 8 + (e + d) % 8
                dst = (d - 1) * HE_C + h * 8 + e
                P[src, dst] = 1.0
    return P


_PERM_NP = _perm_matrix_np()


def _tri_bias_np(n):
    b = np.zeros((2, n, n), dtype=np.float32)
    r = np.arange(n)
    b[0][r[:, None] < r[None, :]] = -1e30
    return b


_TRI_NP = _tri_bias_np(512)


def _top2_gates(logits, p_mat):
    g = jax.nn.sigmoid(logits)
    rows = g.shape[0]
    p_all = lax.dot_general(g, p_mat, (((1,), (0,)), ((), ())),
                            precision=lax.Precision.HIGHEST,
                            preferred_element_type=_F32)
    e_idx = lax.broadcasted_iota(jnp.int32, (rows, HE_C), 1) & 7
    rank = jnp.zeros((rows, HE_C), dtype=jnp.int32)
    for d in range(1, 8):
        p = p_all[:, (d - 1) * HE_C:d * HE_C]
        partner_e = (e_idx + d) & 7
        beats = (p > g) | ((p == g) & (partner_e < e_idx))
        rank = rank + beats.astype(jnp.int32)
    return g * (rank < 2).astype(_F32)


def _proj_gates_body(x_ref, wq_ref, wk_ref, svw_ref, sow_ref, p_ref,
                     q_ref, k_ref, gv_ref, go_ref):
    xb = x_ref[...]
    qb = lax.dot_general(xb, wq_ref[...], (((1,), (1,)), ((), ())),
                         preferred_element_type=_F32)
    kb = lax.dot_general(xb, wk_ref[...], (((1,), (1,)), ((), ())),
                         preferred_element_type=_F32)
    sv = lax.dot_general(xb, svw_ref[...], (((1,), (1,)), ((), ())),
                         preferred_element_type=_F32)
    so = lax.dot_general(xb, sow_ref[...], (((1,), (1,)), ((), ())),
                         preferred_element_type=_F32)
    p_mat = p_ref[...]
    gvb = _top2_gates(sv, p_mat)
    go_ref[...] = _top2_gates(so, p_mat)
    qscale = math.log2(math.e) / math.sqrt(D_HEAD_C)
    for h in range(N_HEADS_C):
        q_ref[h] = qb[:, h * D_HEAD_C:(h + 1) * D_HEAD_C] * qscale
        k_ref[h] = kb[:, h * D_HEAD_C:(h + 1) * D_HEAD_C]
        gv_ref[h] = gvb[:, h * 8:(h + 1) * 8]


VBLK = 512
ABLK = 512
DVA = 128


def _v_body(x_ref, wv_ref, gv_ref, v_ref):
    gvh = gv_ref[0]
    onecol = (lax.broadcasted_iota(jnp.int32, (VBLK, DVA - D_HEAD_C), 1)
              == 0).astype(_F32)
    for si in range(SEQ_C // VBLK):
        xb = x_ref[si * VBLK:(si + 1) * VBLK, :]
        t = lax.dot_general(xb, wv_ref[...], (((1,), (0,)), ((), ())),
                            preferred_element_type=_F32)
        acc = jnp.zeros((VBLK, D_HEAD_C), dtype=_F32)
        for e in range(N_EXPERTS_C):
            gcol = gvh[si * VBLK:(si + 1) * VBLK, e:e + 1]
            acc = acc + gcol * t[:, e * D_HEAD_C:(e + 1) * D_HEAD_C]
        v_ref[0, si * VBLK:(si + 1) * VBLK, 0:D_HEAD_C] = acc
        v_ref[0, si * VBLK:(si + 1) * VBLK, D_HEAD_C:DVA] = onecol


def _attn_body(q_ref, k_ref, v_ref, b_ref, o_ref, acc_sc):
    qi = pl.program_id(1)
    kj = pl.program_id(2)
    nk = pl.num_programs(2)

    @pl.when(kj <= qi)
    def _compute():
        s = lax.dot_general(q_ref[0], k_ref[0], (((1,), (1,)), ((), ())),
                            preferred_element_type=_F32)
        p = jnp.exp2(s + b_ref[0])
        pv = lax.dot_general(p, v_ref[0], (((1,), (0,)), ((), ())),
                             preferred_element_type=_F32)

        @pl.when(kj == 0)
        def _first():
            acc_sc[...] = pv

        @pl.when(kj > 0)
        def _rest():
            acc_sc[...] = acc_sc[...] + pv

    @pl.when(kj == nk - 1)
    def _fin():
        acc = acc_sc[...]
        o_ref[0] = acc[:, 0:D_HEAD_C] / acc[:, D_HEAD_C:D_HEAD_C + 1]


def _out_body(res_ref, go_ref, wo_ref, out_ref, scratch):
    go = go_ref[...]
    for h in range(N_HEADS_C):
        rh = res_ref[h]
        for e in range(N_EXPERTS_C):
            he = h * 8 + e
            scratch[:, he * D_HEAD_C:(he + 1) * D_HEAD_C] = (
                go[:, he:he + 1] * rh)
    out_ref[...] = lax.dot_general(scratch[...], wo_ref[...],
                                   (((1,), (0,)), ((), ())),
                                   preferred_element_type=_F32)


@jax.jit
def kernel(x, Wq, Wk, Wv, Wo, sel_v_w, sel_o_w):
    B, S, D = x.shape
    x2d = x.reshape(S, D)
    p_mat = jnp.asarray(_PERM_NP)

    nsb = S // SBLK
    H, E, dh = N_HEADS_C, N_EXPERTS_C, D_HEAD_C

    q, k, gv, go = pl.pallas_call(
        _proj_gates_body,
        grid=(nsb,),
        in_specs=[
            pl.BlockSpec((SBLK, D), lambda i: (i, 0)),
            pl.BlockSpec((D, D), lambda i: (0, 0)),
            pl.BlockSpec((D, D), lambda i: (0, 0)),
            pl.BlockSpec((HE_C, D), lambda i: (0, 0)),
            pl.BlockSpec((HE_C, D), lambda i: (0, 0)),
            pl.BlockSpec((HE_C, 7 * HE_C), lambda i: (0, 0)),
        ],
        out_specs=[
            pl.BlockSpec((H, SBLK, dh), lambda i: (0, i, 0)),
            pl.BlockSpec((H, SBLK, dh), lambda i: (0, i, 0)),
            pl.BlockSpec((H, SBLK, E), lambda i: (0, i, 0)),
            pl.BlockSpec((SBLK, HE_C), lambda i: (i, 0)),
        ],
        out_shape=[
            jax.ShapeDtypeStruct((H, S, dh), _F32),
            jax.ShapeDtypeStruct((H, S, dh), _F32),
            jax.ShapeDtypeStruct((H, S, E), _F32),
            jax.ShapeDtypeStruct((S, HE_C), _F32),
        ],
    )(x2d, Wq, Wk, sel_v_w, sel_o_w, p_mat)

    wv2d = Wv.reshape(H, E, D, dh).transpose(2, 0, 1, 3).reshape(D, H * E * dh)
    v = pl.pallas_call(
        _v_body,
        grid=(H,),
        in_specs=[
            pl.BlockSpec((S, D), lambda h: (0, 0)),
            pl.BlockSpec((D, E * dh), lambda h: (0, h)),
            pl.BlockSpec((1, S, E), lambda h: (h, 0, 0)),
        ],
        out_specs=pl.BlockSpec((1, S, DVA), lambda h: (h, 0, 0)),
        out_shape=jax.ShapeDtypeStruct((H, S, DVA), _F32),
    )(x2d, wv2d, gv)

    nab = S // ABLK
    tri = jnp.asarray(_TRI_NP)
    res = pl.pallas_call(
        _attn_body,
        grid=(H, nab, nab),
        in_specs=[
            pl.BlockSpec((1, ABLK, dh), lambda h, qi, kj: (h, qi, 0)),
            pl.BlockSpec((1, ABLK, dh),
                         lambda h, qi, kj: (h, jnp.minimum(kj, qi), 0)),
            pl.BlockSpec((1, ABLK, DVA),
                         lambda h, qi, kj: (h, jnp.minimum(kj, qi), 0)),
            pl.BlockSpec((1, ABLK, ABLK),
                         lambda h, qi, kj: (jnp.minimum(jnp.abs(qi - kj), 1),
                                            0, 0)),
        ],
        out_specs=pl.BlockSpec((1, ABLK, dh), lambda h, qi, kj: (h, qi, 0)),
        out_shape=jax.ShapeDtypeStruct((H, S, dh), _F32),
        scratch_shapes=[
            pltpu.VMEM((ABLK, DVA), _F32),
        ],
    )(q, k, v, tri)

    wo2d = Wo.reshape(HE_C * dh, D)
    out2d = pl.pallas_call(
        _out_body,
        grid=(nsb,),
        in_specs=[
            pl.BlockSpec((H, SBLK, dh), lambda i: (0, i, 0)),
            pl.BlockSpec((SBLK, HE_C), lambda i: (i, 0)),
            pl.BlockSpec((HE_C * dh, D), lambda i: (0, 0)),
        ],
        out_specs=pl.BlockSpec((SBLK, D), lambda i: (i, 0)),
        out_shape=jax.ShapeDtypeStruct((S, D), _F32),
        scratch_shapes=[pltpu.VMEM((SBLK, HE_C * dh), _F32)],
    )(res, go, wo2d)

    return out2d.reshape(B, S, D)


# speedup vs baseline: 2.7296x; 1.0184x over previous
import functools
import math

import jax
import jax.numpy as jnp
import numpy as np
from jax import lax
from jax.experimental import pallas as pl
from jax.experimental.pallas import tpu as pltpu

D_MODEL_C = 768
N_HEADS_C = 12
N_EXPERTS_C = 8
D_HEAD_C = 64
HE_C = N_HEADS_C * N_EXPERTS_C
SEQ_C = 2048
SBLK = 256

_F32 = jnp.float32


def _perm_matrix_np():
    P = np.zeros((HE_C, 7 * HE_C), dtype=np.float32)
    for d in range(1, 8):
        for h in range(N_HEADS_C):
            for e in range(N_EXPERTS_C):
                src = h * 8 + (e + d) % 8
                dst = (d - 1) * HE_C + h * 8 + e
                P[src, dst] = 1.0
    return P


_PERM_NP = _perm_matrix_np()


def _tri_bias_np(n):
    b = np.zeros((2, n, n), dtype=np.float32)
    r = np.arange(n)
    b[0][r[:, None] < r[None, :]] = -1e30
    return b


_TRI_NP = _tri_bias_np(512)


def _gate_expand_np():
    M = np.zeros((HE_C, N_EXPERTS_C * D_MODEL_C), dtype=np.float32)
    for h in range(N_HEADS_C):
        for e in range(N_EXPERTS_C):
            base = e * D_MODEL_C + h * D_HEAD_C
            M[h * 8 + e, base:base + D_HEAD_C] = 1.0
    return M


_GEXP_NP = _gate_expand_np()


def _top2_gates(logits, p_mat):
    g = jax.nn.sigmoid(logits)
    rows = g.shape[0]
    p_all = lax.dot_general(g, p_mat, (((1,), (0,)), ((), ())),
                            precision=lax.Precision.HIGHEST,
                            preferred_element_type=_F32)
    e_idx = lax.broadcasted_iota(jnp.int32, (rows, HE_C), 1) & 7
    rank = jnp.zeros((rows, HE_C), dtype=jnp.int32)
    for d in range(1, 8):
        p = p_all[:, (d - 1) * HE_C:d * HE_C]
        partner_e = (e_idx + d) & 7
        beats = (p > g) | ((p == g) & (partner_e < e_idx))
        rank = rank + beats.astype(jnp.int32)
    return g * (rank < 2).astype(_F32)


def _proj_gates_body(x_ref, wq_ref, wk_ref, svw_ref, sow_ref, p_ref,
                     q_ref, k_ref, gv_ref, go_ref):
    xb = x_ref[...]
    qb = lax.dot_general(xb, wq_ref[...], (((1,), (1,)), ((), ())),
                         preferred_element_type=_F32)
    kb = lax.dot_general(xb, wk_ref[...], (((1,), (1,)), ((), ())),
                         preferred_element_type=_F32)
    sv = lax.dot_general(xb, svw_ref[...], (((1,), (1,)), ((), ())),
                         preferred_element_type=_F32)
    so = lax.dot_general(xb, sow_ref[...], (((1,), (1,)), ((), ())),
                         preferred_element_type=_F32)
    p_mat = p_ref[...]
    gvb = _top2_gates(sv, p_mat)
    go_ref[...] = _top2_gates(so, p_mat)
    qscale = math.log2(math.e) / math.sqrt(D_HEAD_C)
    for h in range(N_HEADS_C):
        q_ref[h] = qb[:, h * D_HEAD_C:(h + 1) * D_HEAD_C] * qscale
        k_ref[h] = kb[:, h * D_HEAD_C:(h + 1) * D_HEAD_C]
        gv_ref[h] = gvb[:, h * 8:(h + 1) * 8]


VBLK = 512
ABLK = 512
DVA = 128


def _v_body(x_ref, wv_ref, gv_ref, v_ref):
    gvh = gv_ref[0]
    onecol = (lax.broadcasted_iota(jnp.int32, (VBLK, DVA - D_HEAD_C), 1)
              == 0).astype(_F32)
    for si in range(SEQ_C // VBLK):
        xb = x_ref[si * VBLK:(si + 1) * VBLK, :]
        t = lax.dot_general(xb, wv_ref[...], (((1,), (0,)), ((), ())),
                            preferred_element_type=_F32)
        acc = jnp.zeros((VBLK, D_HEAD_C), dtype=_F32)
        for e in range(N_EXPERTS_C):
            gcol = gvh[si * VBLK:(si + 1) * VBLK, e:e + 1]
            acc = acc + gcol * t[:, e * D_HEAD_C:(e + 1) * D_HEAD_C]
        v_ref[0, si * VBLK:(si + 1) * VBLK, 0:D_HEAD_C] = acc
        v_ref[0, si * VBLK:(si + 1) * VBLK, D_HEAD_C:DVA] = onecol


def _attn_body(q_ref, k_ref, v_ref, b_ref, o_ref, acc_sc):
    qi = pl.program_id(1)
    kj = pl.program_id(2)
    nk = pl.num_programs(2)

    @pl.when(kj <= qi)
    def _compute():
        s = lax.dot_general(q_ref[0], k_ref[0], (((1,), (1,)), ((), ())),
                            preferred_element_type=_F32)
        p = jnp.exp2(s + b_ref[0])
        pv = lax.dot_general(p, v_ref[0], (((1,), (0,)), ((), ())),
                             preferred_element_type=_F32)

        @pl.when(kj == 0)
        def _first():
            acc_sc[...] = pv

        @pl.when(kj > 0)
        def _rest():
            acc_sc[...] = acc_sc[...] + pv

    @pl.when(kj == nk - 1)
    def _fin():
        acc = acc_sc[...]
        o_ref[0] = acc[:, 0:D_HEAD_C] / acc[:, D_HEAD_C:D_HEAD_C + 1]


OBLK = 512


def _out_body(res_ref, go_ref, gexp_ref, wo_ref, out_ref):
    res2 = res_ref[...]
    ge = lax.dot_general(go_ref[...], gexp_ref[...], (((1,), (0,)), ((), ())),
                         preferred_element_type=_F32)
    acc = jnp.zeros((OBLK, D_MODEL_C), dtype=_F32)
    for e in range(N_EXPERTS_C):
        prod = res2 * ge[:, e * D_MODEL_C:(e + 1) * D_MODEL_C]
        acc = acc + lax.dot_general(
            prod, wo_ref[e * D_MODEL_C:(e + 1) * D_MODEL_C, :],
            (((1,), (0,)), ((), ())), preferred_element_type=_F32)
    out_ref[...] = acc


@jax.jit
def kernel(x, Wq, Wk, Wv, Wo, sel_v_w, sel_o_w):
    B, S, D = x.shape
    x2d = x.reshape(S, D)
    p_mat = jnp.asarray(_PERM_NP)

    nsb = S // SBLK
    H, E, dh = N_HEADS_C, N_EXPERTS_C, D_HEAD_C

    q, k, gv, go = pl.pallas_call(
        _proj_gates_body,
        grid=(nsb,),
        in_specs=[
            pl.BlockSpec((SBLK, D), lambda i: (i, 0)),
            pl.BlockSpec((D, D), lambda i: (0, 0)),
            pl.BlockSpec((D, D), lambda i: (0, 0)),
            pl.BlockSpec((HE_C, D), lambda i: (0, 0)),
            pl.BlockSpec((HE_C, D), lambda i: (0, 0)),
            pl.BlockSpec((HE_C, 7 * HE_C), lambda i: (0, 0)),
        ],
        out_specs=[
            pl.BlockSpec((H, SBLK, dh), lambda i: (0, i, 0)),
            pl.BlockSpec((H, SBLK, dh), lambda i: (0, i, 0)),
            pl.BlockSpec((H, SBLK, E), lambda i: (0, i, 0)),
            pl.BlockSpec((SBLK, HE_C), lambda i: (i, 0)),
        ],
        out_shape=[
            jax.ShapeDtypeStruct((H, S, dh), _F32),
            jax.ShapeDtypeStruct((H, S, dh), _F32),
            jax.ShapeDtypeStruct((H, S, E), _F32),
            jax.ShapeDtypeStruct((S, HE_C), _F32),
        ],
    )(x2d, Wq, Wk, sel_v_w, sel_o_w, p_mat)

    wv2d = Wv.reshape(H, E, D, dh).transpose(2, 0, 1, 3).reshape(D, H * E * dh)
    v = pl.pallas_call(
        _v_body,
        grid=(H,),
        in_specs=[
            pl.BlockSpec((S, D), lambda h: (0, 0)),
            pl.BlockSpec((D, E * dh), lambda h: (0, h)),
            pl.BlockSpec((1, S, E), lambda h: (h, 0, 0)),
        ],
        out_specs=pl.BlockSpec((1, S, DVA), lambda h: (h, 0, 0)),
        out_shape=jax.ShapeDtypeStruct((H, S, DVA), _F32),
    )(x2d, wv2d, gv)

    nab = S // ABLK
    tri = jnp.asarray(_TRI_NP)
    res = pl.pallas_call(
        _attn_body,
        grid=(H, nab, nab),
        in_specs=[
            pl.BlockSpec((1, ABLK, dh), lambda h, qi, kj: (h, qi, 0)),
            pl.BlockSpec((1, ABLK, dh),
                         lambda h, qi, kj: (h, jnp.minimum(kj, qi), 0)),
            pl.BlockSpec((1, ABLK, DVA),
                         lambda h, qi, kj: (h, jnp.minimum(kj, qi), 0)),
            pl.BlockSpec((1, ABLK, ABLK),
                         lambda h, qi, kj: (jnp.minimum(jnp.abs(qi - kj), 1),
                                            0, 0)),
        ],
        out_specs=pl.BlockSpec((1, ABLK, dh), lambda h, qi, kj: (h, qi, 0)),
        out_shape=jax.ShapeDtypeStruct((H, S, dh), _F32),
        scratch_shapes=[
            pltpu.VMEM((ABLK, DVA), _F32),
        ],
    )(q, k, v, tri)

    res2d = res.transpose(1, 0, 2).reshape(S, H * dh)
    wo2d = Wo.reshape(H, E, dh, D).transpose(1, 0, 2, 3).reshape(E * dh * H, D)
    gexp = jnp.asarray(_GEXP_NP)
    out2d = pl.pallas_call(
        _out_body,
        grid=(S // OBLK,),
        in_specs=[
            pl.BlockSpec((OBLK, H * dh), lambda i: (i, 0)),
            pl.BlockSpec((OBLK, HE_C), lambda i: (i, 0)),
            pl.BlockSpec((HE_C, E * D), lambda i: (0, 0)),
            pl.BlockSpec((E * dh * H, D), lambda i: (0, 0)),
        ],
        out_specs=pl.BlockSpec((OBLK, D), lambda i: (i, 0)),
        out_shape=jax.ShapeDtypeStruct((S, D), _F32),
    )(res2d, go, gexp, wo2d)

    return out2d.reshape(B, S, D)


# speedup vs baseline: 3.1828x; 1.1660x over previous
import functools
import math

import jax
import jax.numpy as jnp
import numpy as np
from jax import lax
from jax.experimental import pallas as pl
from jax.experimental.pallas import tpu as pltpu

D_MODEL_C = 768
N_HEADS_C = 12
N_EXPERTS_C = 8
D_HEAD_C = 64
HE_C = N_HEADS_C * N_EXPERTS_C
SEQ_C = 2048
SBLK = 256

_F32 = jnp.float32


def _perm_matrix_np():
    P = np.zeros((HE_C, 7 * HE_C), dtype=np.float32)
    for d in range(1, 8):
        for h in range(N_HEADS_C):
            for e in range(N_EXPERTS_C):
                src = h * 8 + (e + d) % 8
                dst = (d - 1) * HE_C + h * 8 + e
                P[src, dst] = 1.0
    return P


_PERM_NP = _perm_matrix_np()


def _tri_bias_np(n):
    b = np.zeros((2, n, n), dtype=np.float32)
    r = np.arange(n)
    b[0][r[:, None] < r[None, :]] = -1e30
    return b


_TRI_NP = _tri_bias_np(512)


def _gate_expand_np():
    M = np.zeros((HE_C, N_EXPERTS_C * D_MODEL_C), dtype=np.float32)
    for h in range(N_HEADS_C):
        for e in range(N_EXPERTS_C):
            base = e * D_MODEL_C + h * D_HEAD_C
            M[h * 8 + e, base:base + D_HEAD_C] = 1.0
    return M


_GEXP_NP = _gate_expand_np()


def _top2_gates(logits, p_mat):
    g = jax.nn.sigmoid(logits)
    rows = g.shape[0]
    p_all = lax.dot_general(g, p_mat, (((1,), (0,)), ((), ())),
                            precision=lax.Precision.HIGHEST,
                            preferred_element_type=_F32)
    e_idx = lax.broadcasted_iota(jnp.int32, (rows, HE_C), 1) & 7
    rank = jnp.zeros((rows, HE_C), dtype=jnp.int32)
    for d in range(1, 8):
        p = p_all[:, (d - 1) * HE_C:d * HE_C]
        partner_e = (e_idx + d) & 7
        beats = (p > g) | ((p == g) & (partner_e < e_idx))
        rank = rank + beats.astype(jnp.int32)
    return g * (rank < 2).astype(_F32)


def _proj_gates_body(x_ref, wq_ref, wk_ref, svw_ref, sow_ref, p_ref,
                     q_ref, k_ref, gv_ref, go_ref):
    xb = x_ref[...]
    qb = lax.dot_general(xb, wq_ref[...], (((1,), (1,)), ((), ())),
                         preferred_element_type=_F32)
    kb = lax.dot_general(xb, wk_ref[...], (((1,), (1,)), ((), ())),
                         preferred_element_type=_F32)
    sv = lax.dot_general(xb, svw_ref[...], (((1,), (1,)), ((), ())),
                         preferred_element_type=_F32)
    so = lax.dot_general(xb, sow_ref[...], (((1,), (1,)), ((), ())),
                         preferred_element_type=_F32)
    p_mat = p_ref[...]
    gvb = _top2_gates(sv, p_mat)
    go_ref[...] = _top2_gates(so, p_mat)
    qscale = math.log2(math.e) / math.sqrt(D_HEAD_C)
    for h in range(N_HEADS_C):
        q_ref[h] = qb[:, h * D_HEAD_C:(h + 1) * D_HEAD_C] * qscale
        k_ref[h] = kb[:, h * D_HEAD_C:(h + 1) * D_HEAD_C]
        gv_ref[h] = gvb[:, h * 8:(h + 1) * 8]


VBLK = 512
ABLK = 512
DVA = 128


def _v_body(x_ref, wv_ref, gv_ref, v_ref):
    gvh = gv_ref[0]
    onecol = (lax.broadcasted_iota(jnp.int32, (VBLK, DVA - D_HEAD_C), 1)
              == 0).astype(_F32)
    for si in range(SEQ_C // VBLK):
        xb = x_ref[si * VBLK:(si + 1) * VBLK, :]
        t = lax.dot_general(xb, wv_ref[...], (((1,), (0,)), ((), ())),
                            preferred_element_type=_F32)
        acc = jnp.zeros((VBLK, D_HEAD_C), dtype=_F32)
        for e in range(N_EXPERTS_C):
            gcol = gvh[si * VBLK:(si + 1) * VBLK, e:e + 1]
            acc = acc + gcol * t[:, e * D_HEAD_C:(e + 1) * D_HEAD_C]
        v_ref[0, si * VBLK:(si + 1) * VBLK, 0:D_HEAD_C] = acc
        v_ref[0, si * VBLK:(si + 1) * VBLK, D_HEAD_C:DVA] = onecol


HPB = 2


def _attn_body(q_ref, k_ref, v_ref, b_ref, o_ref, acc_sc):
    qi = pl.program_id(1)
    kj = pl.program_id(2)
    nk = pl.num_programs(2)

    @pl.when(kj <= qi)
    def _compute():
        bias = b_ref[0]
        for hh in range(HPB):
            s = lax.dot_general(q_ref[hh], k_ref[hh],
                                (((1,), (1,)), ((), ())),
                                preferred_element_type=_F32)
            p = jnp.exp2(s + bias)
            pv = lax.dot_general(p, v_ref[hh], (((1,), (0,)), ((), ())),
                                 preferred_element_type=_F32)

            @pl.when(kj == 0)
            def _first():
                acc_sc[hh] = pv

            @pl.when(kj > 0)
            def _rest():
                acc_sc[hh] = acc_sc[hh] + pv

    @pl.when(kj == nk - 1)
    def _fin():
        for hh in range(HPB):
            acc = acc_sc[hh]
            o_ref[hh] = acc[:, 0:D_HEAD_C] / acc[:, D_HEAD_C:D_HEAD_C + 1]


OBLK = 512


def _out_body(res_ref, go_ref, gexp_ref, wo_ref, out_ref):
    res2 = res_ref[...]
    ge = lax.dot_general(go_ref[...], gexp_ref[...], (((1,), (0,)), ((), ())),
                         preferred_element_type=_F32)
    acc = jnp.zeros((OBLK, D_MODEL_C), dtype=_F32)
    for e in range(N_EXPERTS_C):
        prod = res2 * ge[:, e * D_MODEL_C:(e + 1) * D_MODEL_C]
        acc = acc + lax.dot_general(
            prod, wo_ref[e * D_MODEL_C:(e + 1) * D_MODEL_C, :],
            (((1,), (0,)), ((), ())), preferred_element_type=_F32)
    out_ref[...] = acc


@jax.jit
def kernel(x, Wq, Wk, Wv, Wo, sel_v_w, sel_o_w):
    B, S, D = x.shape
    x2d = x.reshape(S, D)
    p_mat = jnp.asarray(_PERM_NP)

    nsb = S // SBLK
    H, E, dh = N_HEADS_C, N_EXPERTS_C, D_HEAD_C

    q, k, gv, go = pl.pallas_call(
        _proj_gates_body,
        grid=(nsb,),
        in_specs=[
            pl.BlockSpec((SBLK, D), lambda i: (i, 0)),
            pl.BlockSpec((D, D), lambda i: (0, 0)),
            pl.BlockSpec((D, D), lambda i: (0, 0)),
            pl.BlockSpec((HE_C, D), lambda i: (0, 0)),
            pl.BlockSpec((HE_C, D), lambda i: (0, 0)),
            pl.BlockSpec((HE_C, 7 * HE_C), lambda i: (0, 0)),
        ],
        out_specs=[
            pl.BlockSpec((H, SBLK, dh), lambda i: (0, i, 0)),
            pl.BlockSpec((H, SBLK, dh), lambda i: (0, i, 0)),
            pl.BlockSpec((H, SBLK, E), lambda i: (0, i, 0)),
            pl.BlockSpec((SBLK, HE_C), lambda i: (i, 0)),
        ],
        out_shape=[
            jax.ShapeDtypeStruct((H, S, dh), _F32),
            jax.ShapeDtypeStruct((H, S, dh), _F32),
            jax.ShapeDtypeStruct((H, S, E), _F32),
            jax.ShapeDtypeStruct((S, HE_C), _F32),
        ],
    )(x2d, Wq, Wk, sel_v_w, sel_o_w, p_mat)

    wv2d = Wv.reshape(H, E, D, dh).transpose(2, 0, 1, 3).reshape(D, H * E * dh)
    v = pl.pallas_call(
        _v_body,
        grid=(H,),
        in_specs=[
            pl.BlockSpec((S, D), lambda h: (0, 0)),
            pl.BlockSpec((D, E * dh), lambda h: (0, h)),
            pl.BlockSpec((1, S, E), lambda h: (h, 0, 0)),
        ],
        out_specs=pl.BlockSpec((1, S, DVA), lambda h: (h, 0, 0)),
        out_shape=jax.ShapeDtypeStruct((H, S, DVA), _F32),
    )(x2d, wv2d, gv)

    nab = S // ABLK
    tri = jnp.asarray(_TRI_NP)
    res = pl.pallas_call(
        _attn_body,
        grid=(H // HPB, nab, nab),
        in_specs=[
            pl.BlockSpec((HPB, ABLK, dh), lambda h, qi, kj: (h, qi, 0)),
            pl.BlockSpec((HPB, ABLK, dh),
                         lambda h, qi, kj: (h, jnp.minimum(kj, qi), 0)),
            pl.BlockSpec((HPB, ABLK, DVA),
                         lambda h, qi, kj: (h, jnp.minimum(kj, qi), 0)),
            pl.BlockSpec((1, ABLK, ABLK),
                         lambda h, qi, kj: (jnp.minimum(jnp.abs(qi - kj), 1),
                                            0, 0)),
        ],
        out_specs=pl.BlockSpec((HPB, ABLK, dh), lambda h, qi, kj: (h, qi, 0)),
        out_shape=jax.ShapeDtypeStruct((H, S, dh), _F32),
        scratch_shapes=[
            pltpu.VMEM((HPB, ABLK, DVA), _F32),
        ],
    )(q, k, v, tri)

    res2d = res.transpose(1, 0, 2).reshape(S, H * dh)
    wo2d = Wo.reshape(H, E, dh, D).transpose(1, 0, 2, 3).reshape(E * dh * H, D)
    gexp = jnp.asarray(_GEXP_NP)
    out2d = pl.pallas_call(
        _out_body,
        grid=(S // OBLK,),
        in_specs=[
            pl.BlockSpec((OBLK, H * dh), lambda i: (i, 0)),
            pl.BlockSpec((OBLK, HE_C), lambda i: (i, 0)),
            pl.BlockSpec((HE_C, E * D), lambda i: (0, 0)),
            pl.BlockSpec((E * dh * H, D), lambda i: (0, 0)),
        ],
        out_specs=pl.BlockSpec((OBLK, D), lambda i: (i, 0)),
        out_shape=jax.ShapeDtypeStruct((S, D), _F32),
    )(res2d, go, gexp, wo2d)

    return out2d.reshape(B, S, D)
